# baseline (speedup 1.0000x reference)
import functools

import jax
import jax.numpy as jnp
import numpy as np
from jax import lax
from jax.experimental import pallas as pl
from jax.experimental.pallas import tpu as pltpu
from jax.experimental.pallas import tpu_sc as plsc

_INFO = plsc.get_sparse_core_info()
_NC = _INFO.num_cores
_NS = _INFO.num_subcores
_L = _INFO.num_lanes
_NW = _NC * _NS

_K = 128


def _emb_body(nchunk, d, scale, idx_hbm, table_hbm, out_hbm, idx_v, rows_v,
              gsem):
    wid = lax.axis_index("s") * _NC + lax.axis_index("c")
    base = wid * (nchunk * _K)
    pltpu.sync_copy(idx_hbm.at[wid], idx_v)

    def chunk(j, carry):
        pltpu.async_copy(table_hbm.at[idx_v.at[j]], rows_v, gsem).wait()

        def scale_row(r, c2):
            for cc in range(d // _L):
                sl = pl.ds(cc * _L, _L)
                rows_v[r, sl] = rows_v[r, sl] * scale
            return c2

        lax.fori_loop(0, _K, scale_row, 0)
        pltpu.sync_copy(rows_v, out_hbm.at[pl.ds(base + j * _K, _K)])
        return carry

    lax.fori_loop(0, nchunk, chunk, 0)


def kernel(input, table):
    b, s = input.shape
    v, d = table.shape
    n = b * s
    assert n % (_NW * _K) == 0 and d % _L == 0
    nchunk = n // (_NW * _K)
    scale = np.float32(np.sqrt(d))

    idx3 = input.reshape(_NW, nchunk, _K).astype(jnp.int32)

    mesh = plsc.VectorSubcoreMesh(core_axis_name="c", subcore_axis_name="s")
    run = pl.kernel(
        functools.partial(_emb_body, nchunk, d, scale),
        mesh=mesh,
        out_type=jax.ShapeDtypeStruct((n, d), jnp.float32),
        scratch_types=[
            pltpu.VMEM((nchunk, _K), jnp.int32),
            pltpu.VMEM((_K, d), jnp.float32),
            pltpu.SemaphoreType.DMA,
        ],
        compiler_params=pltpu.CompilerParams(use_tc_tiling_on_sc=False),
    )
    out = run(idx3, table)
    return out.reshape(b, s, d)


# speedup vs baseline: 1.2075x; 1.2075x over previous
import functools

import jax
import jax.numpy as jnp
import numpy as np
from jax import lax
from jax.experimental import pallas as pl
from jax.experimental.pallas import tpu as pltpu
from jax.experimental.pallas import tpu_sc as plsc

_INFO = plsc.get_sparse_core_info()
_NC = _INFO.num_cores
_NS = _INFO.num_subcores
_L = _INFO.num_lanes
_NW = _NC * _NS

_K = 128
_NB = 4
_RU = 4


def _emb_body(nchunk, d, scale, idx_hbm, table_hbm, out_hbm, idx_v,
              in_v, out_v, *sems):
    gsems = sems[:_NB]
    osems = sems[_NB:]
    wid = lax.axis_index("s") * _NC + lax.axis_index("c")
    base = wid * (nchunk * _K)
    pltpu.sync_copy(idx_hbm.at[wid], idx_v)

    def start_gather(j, b):
        pltpu.make_async_copy(
            table_hbm.at[idx_v.at[j]], in_v.at[b], gsems[b]).start()

    def wait_gather(j, b):
        pltpu.make_async_copy(
            table_hbm.at[idx_v.at[j]], in_v.at[b], gsems[b]).wait()

    def start_out(j, b):
        pltpu.make_async_copy(
            out_v.at[b], out_hbm.at[pl.ds(base + j * _K, _K)], osems[b]).start()

    def wait_out(j, b):
        pltpu.make_async_copy(
            out_v.at[b], out_hbm.at[pl.ds(base + j * _K, _K)], osems[b]).wait()

    def scale_chunk(b):
        def body(i, carry):
            r0 = i * _RU
            for rr in range(_RU):
                for cc in range(d // _L):
                    sl = pl.ds(cc * _L, _L)
                    out_v[b, r0 + rr, sl] = in_v[b, r0 + rr, sl] * scale
            return carry
        lax.fori_loop(0, _K // _RU, body, 0)

    ngroup = nchunk // _NB

    for b in range(_NB):
        start_gather(b, b)

    for b in range(_NB):
        wait_gather(b, b)
        scale_chunk(b)
        start_out(b, b)
        start_gather(_NB + b, b)

    def group(g, carry):
        for b in range(_NB):
            j = g * _NB + b
            wait_gather(j, b)
            wait_out(j - _NB, b)
            scale_chunk(b)
            start_out(j, b)
            start_gather(j + _NB, b)
        return carry

    lax.fori_loop(1, ngroup - 1, group, 0)

    for b in range(_NB):
        j = (ngroup - 1) * _NB + b
        wait_gather(j, b)
        wait_out(j - _NB, b)
        scale_chunk(b)
        start_out(j, b)

    for b in range(_NB):
        wait_out((ngroup - 1) * _NB + b, b)


def kernel(input, table):
    b, s = input.shape
    v, d = table.shape
    n = b * s
    assert n % (_NW * _K) == 0 and d % _L == 0
    nchunk = n // (_NW * _K)
    assert nchunk % _NB == 0 and nchunk // _NB >= 2
    scale = np.float32(np.sqrt(d))

    idx3 = input.reshape(_NW, nchunk, _K).astype(jnp.int32)

    mesh = plsc.VectorSubcoreMesh(core_axis_name="c", subcore_axis_name="s")
    run = pl.kernel(
        functools.partial(_emb_body, nchunk, d, scale),
        mesh=mesh,
        out_type=jax.ShapeDtypeStruct((n, d), jnp.float32),
        scratch_types=[
            pltpu.VMEM((nchunk, _K), jnp.int32),
            pltpu.VMEM((_NB, _K, d), jnp.float32),
            pltpu.VMEM((_NB, _K, d), jnp.float32),
        ] + [pltpu.SemaphoreType.DMA] * (2 * _NB),
        compiler_params=pltpu.CompilerParams(use_tc_tiling_on_sc=False),
    )
    out = run(idx3, table)
    return out.reshape(b, s, d)
